# baseline (speedup 1.0000x reference)
import jax
import jax.numpy as jnp
from jax.experimental import pallas as pl
from jax.experimental.pallas import tpu as pltpu

N_NODES = 50000
HID = 64
BLK = 512
N_PAD = 50176


def _proj_body(x_ref, w_ref, as_ref, ad_ref, h_ref, als_ref, ald_ref):
    h = x_ref[...] @ w_ref[...]
    h_ref[...] = h
    als_ref[...] = h @ as_ref[...]
    ald_ref[...] = h @ ad_ref[...]


def _proj(x, W, a_s, a_d):
    n, d = x.shape
    grid = (n // BLK,)
    return pl.pallas_call(
        _proj_body,
        grid=grid,
        in_specs=[
            pl.BlockSpec((BLK, d), lambda i: (i, 0)),
            pl.BlockSpec((d, HID), lambda i: (0, 0)),
            pl.BlockSpec((HID, 1), lambda i: (0, 0)),
            pl.BlockSpec((HID, 1), lambda i: (0, 0)),
        ],
        out_specs=[
            pl.BlockSpec((BLK, HID), lambda i: (i, 0)),
            pl.BlockSpec((BLK, 1), lambda i: (i, 0)),
            pl.BlockSpec((BLK, 1), lambda i: (i, 0)),
        ],
        out_shape=[
            jax.ShapeDtypeStruct((n, HID), jnp.float32),
            jax.ShapeDtypeStruct((n, 1), jnp.float32),
            jax.ShapeDtypeStruct((n, 1), jnp.float32),
        ],
    )(x, W, a_s.reshape(HID, 1), a_d.reshape(HID, 1))


def _gat_layer(x, src, dst, edge_attr, W, a_s, a_d, We, a_e, b):
    n = x.shape[0]
    xp = jnp.pad(x, ((0, N_PAD - n), (0, 0)))
    hp, alsp, aldp = _proj(xp, W, a_s, a_d)
    h = hp[:n]
    alpha_src = alsp[:n, 0]
    alpha_dst = aldp[:n, 0]

    loop = jnp.arange(n)
    src_f = jnp.concatenate([src, loop])
    dst_f = jnp.concatenate([dst, loop])
    ea_loop = jnp.broadcast_to(edge_attr.mean(axis=0), (n, edge_attr.shape[1]))
    ea_f = jnp.concatenate([edge_attr, ea_loop], axis=0)
    alpha_e = (ea_f @ We) @ a_e
    alpha = alpha_src[src_f] + alpha_dst[dst_f] + alpha_e
    alpha = jax.nn.leaky_relu(alpha, 0.2)
    amax = jax.ops.segment_max(alpha, dst_f, num_segments=n)
    ex = jnp.exp(alpha - amax[dst_f])
    denom = jax.ops.segment_sum(ex, dst_f, num_segments=n)
    coef = ex / (denom[dst_f] + 1e-16)
    out = jax.ops.segment_sum(h[src_f] * coef[:, None], dst_f, num_segments=n)
    return out + b


def kernel(x, edge_index, edge_attr, W0, as0, ad0, We0, ae0, b0, W1, as1, ad1, We1, ae1, b1, W2, as2, ad2, We2, ae2, b2, imp_W1, imp_b1, imp_W2, imp_b2, ct_W1, ct_b1, ct_W2, ct_b2, tm_W1, tm_b1, tm_W2, tm_b2):
    src = edge_index[0]
    dst = edge_index[1]
    h = _gat_layer(x, src, dst, edge_attr, W0, as0, ad0, We0, ae0, b0)
    h = jax.nn.relu(h)
    h = _gat_layer(h, src, dst, edge_attr, W1, as1, ad1, We1, ae1, b1)
    h = jax.nn.relu(h)
    h = _gat_layer(h, src, dst, edge_attr, W2, as2, ad2, We2, ae2, b2)
    impedance = jax.nn.relu(h @ imp_W1 + imp_b1) @ imp_W2 + imp_b2
    pair = jnp.concatenate([h[src], h[dst]], axis=-1)
    crosstalk = (jax.nn.relu(pair @ ct_W1 + ct_b1) @ ct_W2 + ct_b2)[:, 0]
    timing = jax.nn.relu(h @ tm_W1 + tm_b1) @ tm_W2 + tm_b2
    return (impedance, crosstalk, timing)


# speedup vs baseline: 9.7373x; 9.7373x over previous
import functools

import jax
import jax.numpy as jnp
from jax import lax
from jax.experimental import pallas as pl
from jax.experimental.pallas import tpu as pltpu
from jax.experimental.pallas import tpu_sc as plsc

N = 50000
E = 800000
H = 64
HH = 32
BLK = 1000
NC = 2
NS = 16
LANES = 128
EF_ROWS = 6656
EF = EF_ROWS * LANES
TROWS = EF_ROWS // NS
SUPER = 8
NSUPER = TROWS // SUPER
XT_ROWS = 6272
XTT = XT_ROWS // (NC * NS)
NPAD = 50048
NSLICE = NPAD // NS

_mesh = plsc.VectorSubcoreMesh(core_axis_name="c", subcore_axis_name="s")


def _relu(v):
    return jnp.maximum(v, 0.0)


def _edge_alpha_body(ea_ref, v_ref, o_ref, s_ref, m_ref):
    i = pl.program_id(0)
    a3 = ea_ref[...] @ v_ref[...]

    @pl.when(i == 0)
    def _():
        s_ref[...] = jnp.zeros_like(s_ref)
        m_ref[...] = jnp.full_like(m_ref, -1e30)

    o_ref[...] = a3
    s_ref[...] = s_ref[...] + jnp.sum(a3, axis=0, keepdims=True)
    m_ref[...] = jnp.maximum(m_ref[...], jnp.max(a3, axis=0, keepdims=True))


def _edge_alpha(ea, vw):
    grid = (E // BLK,)
    return pl.pallas_call(
        _edge_alpha_body,
        grid=grid,
        in_specs=[
            pl.BlockSpec((BLK, 2), lambda i: (i, 0)),
            pl.BlockSpec((2, 4), lambda i: (0, 0)),
        ],
        out_specs=[
            pl.BlockSpec((BLK, 4), lambda i: (i, 0)),
            pl.BlockSpec((1, 4), lambda i: (0, 0)),
            pl.BlockSpec((1, 4), lambda i: (0, 0)),
        ],
        out_shape=[
            jax.ShapeDtypeStruct((E, 4), jnp.float32),
            jax.ShapeDtypeStruct((1, 4), jnp.float32),
            jax.ShapeDtypeStruct((1, 4), jnp.float32),
        ],
    )(ea, vw)


def _proj_tail(h, as_ref, ad_ref, hl_ref, hr_ref, a_ref, d_ref, ms_ref, md_ref, i):
    hl_ref[...] = h[:, :HH]
    hr_ref[...] = h[:, HH:]
    als = h @ as_ref[...]
    ald = h @ ad_ref[...]
    a_ref[...] = als
    d_ref[...] = ald

    @pl.when(i == 0)
    def _():
        ms_ref[...] = jnp.full_like(ms_ref, -1e30)
        md_ref[...] = jnp.full_like(md_ref, -1e30)

    ms_ref[...] = jnp.maximum(ms_ref[...], jnp.max(als, axis=0, keepdims=True))
    md_ref[...] = jnp.maximum(md_ref[...], jnp.max(ald, axis=0, keepdims=True))


def _proj0_body(x_ref, w_ref, as_ref, ad_ref,
                hl_ref, hr_ref, a_ref, d_ref, ms_ref, md_ref):
    h = x_ref[...] @ w_ref[...]
    _proj_tail(h, as_ref, ad_ref, hl_ref, hr_ref, a_ref, d_ref, ms_ref, md_ref,
               pl.program_id(0))


def _proj_mid_body(al_ref, ar_ref, dn_ref, b_ref, w_ref, as_ref, ad_ref,
                   hl_ref, hr_ref, a_ref, d_ref, ms_ref, md_ref):
    invd = 1.0 / (dn_ref[...] + 1e-16)
    xl = _relu(al_ref[...] * invd + b_ref[:, :HH])
    xr = _relu(ar_ref[...] * invd + b_ref[:, HH:])
    h = xl @ w_ref[:HH, :] + xr @ w_ref[HH:, :]
    _proj_tail(h, as_ref, ad_ref, hl_ref, hr_ref, a_ref, d_ref, ms_ref, md_ref,
               pl.program_id(0))


def _proj_outs():
    return (
        [
            pl.BlockSpec((BLK, HH), lambda i: (i, 0)),
            pl.BlockSpec((BLK, HH), lambda i: (i, 0)),
            pl.BlockSpec((BLK, 1), lambda i: (i, 0)),
            pl.BlockSpec((BLK, 1), lambda i: (i, 0)),
            pl.BlockSpec((1, 1), lambda i: (0, 0)),
            pl.BlockSpec((1, 1), lambda i: (0, 0)),
        ],
        [
            jax.ShapeDtypeStruct((N, HH), jnp.float32),
            jax.ShapeDtypeStruct((N, HH), jnp.float32),
            jax.ShapeDtypeStruct((N, 1), jnp.float32),
            jax.ShapeDtypeStruct((N, 1), jnp.float32),
            jax.ShapeDtypeStruct((1, 1), jnp.float32),
            jax.ShapeDtypeStruct((1, 1), jnp.float32),
        ],
    )


def _proj0(x, w, a_s, a_d):
    grid = (N // BLK,)
    outs, shapes = _proj_outs()
    return pl.pallas_call(
        _proj0_body,
        grid=grid,
        in_specs=[
            pl.BlockSpec((BLK, 4), lambda i: (i, 0)),
            pl.BlockSpec((4, H), lambda i: (0, 0)),
            pl.BlockSpec((H, 1), lambda i: (0, 0)),
            pl.BlockSpec((H, 1), lambda i: (0, 0)),
        ],
        out_specs=outs,
        out_shape=shapes,
    )(x, w, a_s, a_d)


def _proj_mid(accl, accr, den, b, w, a_s, a_d):
    grid = (N // BLK,)
    outs, shapes = _proj_outs()
    return pl.pallas_call(
        _proj_mid_body,
        grid=grid,
        in_specs=[
            pl.BlockSpec((BLK, HH), lambda i: (i, 0)),
            pl.BlockSpec((BLK, HH), lambda i: (i, 0)),
            pl.BlockSpec((BLK, 1), lambda i: (i, 0)),
            pl.BlockSpec((1, H), lambda i: (0, 0)),
            pl.BlockSpec((H, H), lambda i: (0, 0)),
            pl.BlockSpec((H, 1), lambda i: (0, 0)),
            pl.BlockSpec((H, 1), lambda i: (0, 0)),
        ],
        out_specs=outs,
        out_shape=shapes,
    )(accl, accr, den, b, w, a_s, a_d)


def _final_body(al_ref, ar_ref, dn_ref, b_ref,
                iw1_ref, ib1_ref, iw2_ref, ib2_ref,
                tw1_ref, tb1_ref, tw2_ref, tb2_ref,
                cwp_ref, cwq_ref, cb1_ref,
                imp_ref, tm_ref, p_ref, q_ref):
    invd = 1.0 / (dn_ref[...] + 1e-16)
    hl = al_ref[...] * invd + b_ref[:, :HH]
    hr = ar_ref[...] * invd + b_ref[:, HH:]
    h = jnp.concatenate([hl, hr], axis=1)
    imp_ref[...] = _relu(h @ iw1_ref[...] + ib1_ref[...]) @ iw2_ref[...] + ib2_ref[...]
    tm_ref[...] = _relu(h @ tw1_ref[...] + tb1_ref[...]) @ tw2_ref[...] + tb2_ref[...]
    p_ref[...] = h @ cwp_ref[...] + cb1_ref[...]
    q_ref[...] = h @ cwq_ref[...]


def _final(accl, accr, den, b, iw1, ib1, iw2, ib2, tw1, tb1, tw2, tb2,
           cwp, cwq, cb1):
    grid = (N // BLK,)
    h2 = H // 2
    return pl.pallas_call(
        _final_body,
        grid=grid,
        in_specs=[
            pl.BlockSpec((BLK, HH), lambda i: (i, 0)),
            pl.BlockSpec((BLK, HH), lambda i: (i, 0)),
            pl.BlockSpec((BLK, 1), lambda i: (i, 0)),
            pl.BlockSpec((1, H), lambda i: (0, 0)),
            pl.BlockSpec((H, h2), lambda i: (0, 0)),
            pl.BlockSpec((1, h2), lambda i: (0, 0)),
            pl.BlockSpec((h2, 1), lambda i: (0, 0)),
            pl.BlockSpec((1, 1), lambda i: (0, 0)),
            pl.BlockSpec((H, h2), lambda i: (0, 0)),
            pl.BlockSpec((1, h2), lambda i: (0, 0)),
            pl.BlockSpec((h2, 1), lambda i: (0, 0)),
            pl.BlockSpec((1, 1), lambda i: (0, 0)),
            pl.BlockSpec((H, H), lambda i: (0, 0)),
            pl.BlockSpec((H, H), lambda i: (0, 0)),
            pl.BlockSpec((1, H), lambda i: (0, 0)),
        ],
        out_specs=[
            pl.BlockSpec((BLK, 1), lambda i: (i, 0)),
            pl.BlockSpec((BLK, 1), lambda i: (i, 0)),
            pl.BlockSpec((BLK, H), lambda i: (i, 0)),
            pl.BlockSpec((BLK, H), lambda i: (i, 0)),
        ],
        out_shape=[
            jax.ShapeDtypeStruct((N, 1), jnp.float32),
            jax.ShapeDtypeStruct((N, 1), jnp.float32),
            jax.ShapeDtypeStruct((N, H), jnp.float32),
            jax.ShapeDtypeStruct((N, H), jnp.float32),
        ],
    )(accl, accr, den, b, iw1, ib1, iw2, ib2, tw1, tb1, tw2, tb2, cwp, cwq, cb1)


def _edge_sc_body(as_h, ad_h, ae_h, src_h, dst_h, m_h, hl_h, hr_h, za_h, zd_h,
                  acc_o, den_o,
                  src2, dst2, ae2, av1, dv1, ex1, rowb, mv, acc_s, den_s):
    cid = lax.axis_index("c")
    sid = lax.axis_index("s")

    pltpu.sync_copy(m_h, mv)
    zsl = pl.ds(sid * NSLICE, NSLICE)
    pltpu.sync_copy(za_h, acc_s.at[zsl])

    @pl.when(sid == 0)
    def _():
        pltpu.sync_copy(zd_h, den_s)

    plsc.subcore_barrier()

    mvv = mv[...]
    base_row = sid * TROWS

    def superchunk(ci, carry):
        row0 = base_row + ci * SUPER
        pltpu.sync_copy(src_h.at[pl.ds(row0, SUPER), :], src2)
        pltpu.sync_copy(dst_h.at[pl.ds(row0, SUPER), :], dst2)
        pltpu.sync_copy(ae_h.at[pl.ds(row0, SUPER), :], ae2)

        def edge_row(k, c2):
            pltpu.sync_copy(as_h.at[src2.at[k]], av1)
            pltpu.sync_copy(ad_h.at[dst2.at[k]], dv1)

            @pl.when(cid == 0)
            def _():
                pltpu.sync_copy(hl_h.at[src2.at[k]], rowb)

            @pl.when(cid == 1)
            def _():
                pltpu.sync_copy(hr_h.at[src2.at[k]], rowb)

            def grp(g, c3):
                sl = pl.ds(g * 16, 16)
                s = av1[sl] + dv1[sl] + ae2[k, sl]
                a = jnp.maximum(s, 0.2 * s)
                ex1[sl] = jnp.exp(a - mvv)
                return c3

            lax.fori_loop(0, LANES // 16, grp, 0)

            def scale(g, c3):
                lo = pl.ds(0, 16)
                hi = pl.ds(16, 16)
                for r in range(16):
                    j = g * 16 + r
                    sc = plsc.load_gather(ex1, [jnp.full((16,), j, jnp.int32)])
                    rowb[j, lo] = rowb[j, lo] * sc
                    rowb[j, hi] = rowb[j, hi] * sc
                return c3

            lax.fori_loop(0, LANES // 16, scale, 0)

            pltpu.sync_copy(rowb, acc_s.at[dst2.at[k]], add=True)

            @pl.when(cid == 0)
            def _():
                pltpu.sync_copy(ex1, den_s.at[dst2.at[k]], add=True)
            return c2

        lax.fori_loop(0, SUPER, edge_row, 0)
        return carry

    lax.fori_loop(0, NSUPER, superchunk, 0)
    plsc.subcore_barrier()

    @pl.when(cid == 0)
    def _():
        pltpu.sync_copy(acc_s.at[zsl], acc_o.at[0].at[zsl])

    @pl.when(cid == 1)
    def _():
        pltpu.sync_copy(acc_s.at[zsl], acc_o.at[1].at[zsl])

    @pl.when(jnp.logical_and(cid == 0, sid == 0))
    def _():
        pltpu.sync_copy(den_s, den_o)


_edge_sc = functools.partial(
    pl.kernel,
    _edge_sc_body,
    mesh=_mesh,
    compiler_params=pltpu.CompilerParams(
        needs_layout_passes=False, use_tc_tiling_on_sc=False),
    out_type=[
        jax.ShapeDtypeStruct((2, NPAD, HH), jnp.float32),
        jax.ShapeDtypeStruct((NPAD,), jnp.float32),
    ],
    scratch_types=[
        pltpu.VMEM((SUPER, LANES), jnp.int32),
        pltpu.VMEM((SUPER, LANES), jnp.int32),
        pltpu.VMEM((SUPER, LANES), jnp.float32),
        pltpu.VMEM((LANES,), jnp.float32),
        pltpu.VMEM((LANES,), jnp.float32),
        pltpu.VMEM((LANES,), jnp.float32),
        pltpu.VMEM((LANES, HH), jnp.float32),
        pltpu.VMEM((16,), jnp.float32),
        pltpu.VMEM_SHARED((NPAD, HH), jnp.float32),
        pltpu.VMEM_SHARED((NPAD,), jnp.float32),
    ],
)


def _xt_sc_body(p_h, q_h, src_h, dst_h, w2_h, b2_h,
                ct_o, wexp, b2v, srcb, dstb, bufp, bufq, outb):
    cid = lax.axis_index("c")
    sid = lax.axis_index("s")
    wid = sid * NC + cid
    base = wid * XTT

    pltpu.sync_copy(w2_h, wexp)
    pltpu.sync_copy(b2_h, b2v)
    b2vec = b2v[...]

    def row(r, carry):
        rr = base + r
        pltpu.sync_copy(src_h.at[pl.ds(rr, 1), :], srcb)
        pltpu.sync_copy(dst_h.at[pl.ds(rr, 1), :], dstb)
        pltpu.sync_copy(p_h.at[srcb.at[0]], bufp)
        pltpu.sync_copy(q_h.at[dstb.at[0]], bufq)
        for g in range(LANES // 16):
            rows16 = lax.iota(jnp.int32, 16) + g * 16

            def fbody(f, acc):
                fi = jnp.full((16,), f, jnp.int32)
                pf = plsc.load_gather(bufp, [rows16, fi])
                qf = plsc.load_gather(bufq, [rows16, fi])
                t = jnp.maximum(pf + qf, 0.0)
                return acc + t * wexp[f, pl.ds(0, 16)]

            acc = lax.fori_loop(0, H, fbody, jnp.zeros((16,), jnp.float32))
            outb[pl.ds(g * 16, 16)] = acc + b2vec
        pltpu.sync_copy(outb, ct_o.at[rr])
        return carry

    lax.fori_loop(0, XTT, row, 0)


_xt_sc = functools.partial(
    pl.kernel,
    _xt_sc_body,
    mesh=_mesh,
    compiler_params=pltpu.CompilerParams(
        needs_layout_passes=False, use_tc_tiling_on_sc=False),
    out_type=jax.ShapeDtypeStruct((XT_ROWS, LANES), jnp.float32),
    scratch_types=[
        pltpu.VMEM((H, 16), jnp.float32),
        pltpu.VMEM((16,), jnp.float32),
        pltpu.VMEM((1, LANES), jnp.int32),
        pltpu.VMEM((1, LANES), jnp.int32),
        pltpu.VMEM((LANES, H), jnp.float32),
        pltpu.VMEM((LANES, H), jnp.float32),
        pltpu.VMEM((LANES,), jnp.float32),
    ],
)


def kernel(x, edge_index, edge_attr, W0, as0, ad0, We0, ae0, b0,
           W1, as1, ad1, We1, ae1, b1, W2, as2, ad2, We2, ae2, b2,
           imp_W1, imp_b1, imp_W2, imp_b2, ct_W1, ct_b1, ct_W2, ct_b2,
           tm_W1, tm_b1, tm_W2, tm_b2):
    src = edge_index[0].astype(jnp.int32)
    dst = edge_index[1].astype(jnp.int32)
    loop = jnp.arange(N, dtype=jnp.int32)
    padi = jnp.zeros((EF - E - N,), jnp.int32)
    srcf = jnp.concatenate([src, loop, padi]).reshape(EF_ROWS, LANES)
    dstf = jnp.concatenate([dst, loop, padi]).reshape(EF_ROWS, LANES)

    vw = jnp.stack([We0 @ ae0, We1 @ ae1, We2 @ ae2, jnp.zeros(2)], axis=1)
    ae3, s3, m3 = _edge_alpha(edge_attr, vw.astype(jnp.float32))
    cmean = s3[0] / float(E)

    za = jnp.zeros((NSLICE, HH), jnp.float32)
    zd = jnp.zeros((NPAD,), jnp.float32)
    neg = jnp.full((EF - E - N,), -1e30, jnp.float32)

    def run_layer(proj_out, lidx):
        hl, hr, a_s, a_d, ms, md = proj_out
        mshift = jnp.maximum(0.0, ms[0, 0] + md[0, 0] + m3[0, lidx])
        aef = jnp.concatenate(
            [ae3[:, lidx], jnp.full((N,), cmean[lidx], jnp.float32), neg]
        ).reshape(EF_ROWS, LANES)
        m16 = jnp.broadcast_to(mshift, (16,)).astype(jnp.float32)
        acc, den = _edge_sc()(a_s.reshape(N), a_d.reshape(N), aef, srcf, dstf,
                              m16, hl, hr, za, zd)
        return acc[:, :N, :], den[:N].reshape(N, 1)

    acc, den = run_layer(_proj0(x, W0, as0.reshape(H, 1), ad0.reshape(H, 1)), 0)
    acc, den = run_layer(
        _proj_mid(acc[0], acc[1], den, b0.reshape(1, H), W1,
                  as1.reshape(H, 1), ad1.reshape(H, 1)), 1)
    acc, den = run_layer(
        _proj_mid(acc[0], acc[1], den, b1.reshape(1, H), W2,
                  as2.reshape(H, 1), ad2.reshape(H, 1)), 2)

    h2 = H // 2
    imp, tm, pp, qq = _final(
        acc[0], acc[1], den, b2.reshape(1, H),
        imp_W1, imp_b1.reshape(1, h2), imp_W2, imp_b2.reshape(1, 1),
        tm_W1, tm_b1.reshape(1, h2), tm_W2, tm_b2.reshape(1, 1),
        ct_W1[:H, :], ct_W1[H:, :], ct_b1.reshape(1, H))

    padx = jnp.zeros((XT_ROWS * LANES - E,), jnp.int32)
    srcx = jnp.concatenate([src, padx]).reshape(XT_ROWS, LANES)
    dstx = jnp.concatenate([dst, padx]).reshape(XT_ROWS, LANES)
    b2c = jnp.broadcast_to(ct_b2.reshape(()), (16,)).astype(jnp.float32)
    w2x = jnp.broadcast_to(ct_W2.reshape(H, 1), (H, 16)).astype(jnp.float32)
    ct = _xt_sc()(pp, qq, srcx, dstx, w2x, b2c)
    crosstalk = ct.reshape(-1)[:E]

    return (imp, crosstalk, tm)


# speedup vs baseline: 11.8186x; 1.2137x over previous
import functools

import jax
import jax.numpy as jnp
from jax import lax
from jax.experimental import pallas as pl
from jax.experimental.pallas import tpu as pltpu
from jax.experimental.pallas import tpu_sc as plsc

N = 50000
E = 800000
H = 64
HH = 32
BLK = 1000
NC = 2
NS = 16
LANES = 128
EF_ROWS = 6656
EF = EF_ROWS * LANES
TROWS = EF_ROWS // NS
SUPER = 8
NSUPER = TROWS // SUPER
XT_ROWS = 6272
XTT = XT_ROWS // (NC * NS)
NPAD = 50048
NSLICE = NPAD // NS

_mesh = plsc.VectorSubcoreMesh(core_axis_name="c", subcore_axis_name="s")


def _relu(v):
    return jnp.maximum(v, 0.0)


def _edge_alpha_body(ea_ref, v_ref, o_ref, s_ref, m_ref):
    i = pl.program_id(0)
    a3 = ea_ref[...] @ v_ref[...]

    @pl.when(i == 0)
    def _():
        s_ref[...] = jnp.zeros_like(s_ref)
        m_ref[...] = jnp.full_like(m_ref, -1e30)

    o_ref[...] = a3
    s_ref[...] = s_ref[...] + jnp.sum(a3, axis=0, keepdims=True)
    m_ref[...] = jnp.maximum(m_ref[...], jnp.max(a3, axis=0, keepdims=True))


def _edge_alpha(ea, vw):
    grid = (E // BLK,)
    return pl.pallas_call(
        _edge_alpha_body,
        grid=grid,
        in_specs=[
            pl.BlockSpec((BLK, 2), lambda i: (i, 0)),
            pl.BlockSpec((2, 4), lambda i: (0, 0)),
        ],
        out_specs=[
            pl.BlockSpec((BLK, 4), lambda i: (i, 0)),
            pl.BlockSpec((1, 4), lambda i: (0, 0)),
            pl.BlockSpec((1, 4), lambda i: (0, 0)),
        ],
        out_shape=[
            jax.ShapeDtypeStruct((E, 4), jnp.float32),
            jax.ShapeDtypeStruct((1, 4), jnp.float32),
            jax.ShapeDtypeStruct((1, 4), jnp.float32),
        ],
    )(ea, vw)


def _proj_tail(h, as_ref, ad_ref, hl_ref, hr_ref, a_ref, d_ref, ms_ref, md_ref, i):
    hl_ref[...] = h[:, :HH]
    hr_ref[...] = h[:, HH:]
    als = h @ as_ref[...]
    ald = h @ ad_ref[...]
    a_ref[...] = als
    d_ref[...] = ald

    @pl.when(i == 0)
    def _():
        ms_ref[...] = jnp.full_like(ms_ref, -1e30)
        md_ref[...] = jnp.full_like(md_ref, -1e30)

    ms_ref[...] = jnp.maximum(ms_ref[...], jnp.max(als, axis=0, keepdims=True))
    md_ref[...] = jnp.maximum(md_ref[...], jnp.max(ald, axis=0, keepdims=True))


def _proj0_body(x_ref, w_ref, as_ref, ad_ref,
                hl_ref, hr_ref, a_ref, d_ref, ms_ref, md_ref):
    h = x_ref[...] @ w_ref[...]
    _proj_tail(h, as_ref, ad_ref, hl_ref, hr_ref, a_ref, d_ref, ms_ref, md_ref,
               pl.program_id(0))


def _proj_mid_body(al_ref, ar_ref, dn_ref, b_ref, w_ref, as_ref, ad_ref,
                   hl_ref, hr_ref, a_ref, d_ref, ms_ref, md_ref):
    invd = 1.0 / (dn_ref[...] + 1e-16)
    xl = _relu(al_ref[...] * invd + b_ref[:, :HH])
    xr = _relu(ar_ref[...] * invd + b_ref[:, HH:])
    h = xl @ w_ref[:HH, :] + xr @ w_ref[HH:, :]
    _proj_tail(h, as_ref, ad_ref, hl_ref, hr_ref, a_ref, d_ref, ms_ref, md_ref,
               pl.program_id(0))


def _proj_outs():
    return (
        [
            pl.BlockSpec((BLK, HH), lambda i: (i, 0)),
            pl.BlockSpec((BLK, HH), lambda i: (i, 0)),
            pl.BlockSpec((BLK, 1), lambda i: (i, 0)),
            pl.BlockSpec((BLK, 1), lambda i: (i, 0)),
            pl.BlockSpec((1, 1), lambda i: (0, 0)),
            pl.BlockSpec((1, 1), lambda i: (0, 0)),
        ],
        [
            jax.ShapeDtypeStruct((N, HH), jnp.float32),
            jax.ShapeDtypeStruct((N, HH), jnp.float32),
            jax.ShapeDtypeStruct((N, 1), jnp.float32),
            jax.ShapeDtypeStruct((N, 1), jnp.float32),
            jax.ShapeDtypeStruct((1, 1), jnp.float32),
            jax.ShapeDtypeStruct((1, 1), jnp.float32),
        ],
    )


def _proj0(x, w, a_s, a_d):
    grid = (N // BLK,)
    outs, shapes = _proj_outs()
    return pl.pallas_call(
        _proj0_body,
        grid=grid,
        in_specs=[
            pl.BlockSpec((BLK, 4), lambda i: (i, 0)),
            pl.BlockSpec((4, H), lambda i: (0, 0)),
            pl.BlockSpec((H, 1), lambda i: (0, 0)),
            pl.BlockSpec((H, 1), lambda i: (0, 0)),
        ],
        out_specs=outs,
        out_shape=shapes,
    )(x, w, a_s, a_d)


def _proj_mid(accl, accr, den, b, w, a_s, a_d):
    grid = (N // BLK,)
    outs, shapes = _proj_outs()
    return pl.pallas_call(
        _proj_mid_body,
        grid=grid,
        in_specs=[
            pl.BlockSpec((BLK, HH), lambda i: (i, 0)),
            pl.BlockSpec((BLK, HH), lambda i: (i, 0)),
            pl.BlockSpec((BLK, 1), lambda i: (i, 0)),
            pl.BlockSpec((1, H), lambda i: (0, 0)),
            pl.BlockSpec((H, H), lambda i: (0, 0)),
            pl.BlockSpec((H, 1), lambda i: (0, 0)),
            pl.BlockSpec((H, 1), lambda i: (0, 0)),
        ],
        out_specs=outs,
        out_shape=shapes,
    )(accl, accr, den, b, w, a_s, a_d)


def _final_body(al_ref, ar_ref, dn_ref, b_ref,
                iw1_ref, ib1_ref, iw2_ref, ib2_ref,
                tw1_ref, tb1_ref, tw2_ref, tb2_ref,
                cwp_ref, cwq_ref, cb1_ref,
                imp_ref, tm_ref, p_ref, q_ref):
    invd = 1.0 / (dn_ref[...] + 1e-16)
    hl = al_ref[...] * invd + b_ref[:, :HH]
    hr = ar_ref[...] * invd + b_ref[:, HH:]
    h = jnp.concatenate([hl, hr], axis=1)
    imp_ref[...] = _relu(h @ iw1_ref[...] + ib1_ref[...]) @ iw2_ref[...] + ib2_ref[...]
    tm_ref[...] = _relu(h @ tw1_ref[...] + tb1_ref[...]) @ tw2_ref[...] + tb2_ref[...]
    p_ref[...] = h @ cwp_ref[...] + cb1_ref[...]
    q_ref[...] = h @ cwq_ref[...]


def _final(accl, accr, den, b, iw1, ib1, iw2, ib2, tw1, tb1, tw2, tb2,
           cwp, cwq, cb1):
    grid = (N // BLK,)
    h2 = H // 2
    return pl.pallas_call(
        _final_body,
        grid=grid,
        in_specs=[
            pl.BlockSpec((BLK, HH), lambda i: (i, 0)),
            pl.BlockSpec((BLK, HH), lambda i: (i, 0)),
            pl.BlockSpec((BLK, 1), lambda i: (i, 0)),
            pl.BlockSpec((1, H), lambda i: (0, 0)),
            pl.BlockSpec((H, h2), lambda i: (0, 0)),
            pl.BlockSpec((1, h2), lambda i: (0, 0)),
            pl.BlockSpec((h2, 1), lambda i: (0, 0)),
            pl.BlockSpec((1, 1), lambda i: (0, 0)),
            pl.BlockSpec((H, h2), lambda i: (0, 0)),
            pl.BlockSpec((1, h2), lambda i: (0, 0)),
            pl.BlockSpec((h2, 1), lambda i: (0, 0)),
            pl.BlockSpec((1, 1), lambda i: (0, 0)),
            pl.BlockSpec((H, H), lambda i: (0, 0)),
            pl.BlockSpec((H, H), lambda i: (0, 0)),
            pl.BlockSpec((1, H), lambda i: (0, 0)),
        ],
        out_specs=[
            pl.BlockSpec((BLK, 1), lambda i: (i, 0)),
            pl.BlockSpec((BLK, 1), lambda i: (i, 0)),
            pl.BlockSpec((BLK, H), lambda i: (i, 0)),
            pl.BlockSpec((BLK, H), lambda i: (i, 0)),
        ],
        out_shape=[
            jax.ShapeDtypeStruct((N, 1), jnp.float32),
            jax.ShapeDtypeStruct((N, 1), jnp.float32),
            jax.ShapeDtypeStruct((N, H), jnp.float32),
            jax.ShapeDtypeStruct((N, H), jnp.float32),
        ],
    )(accl, accr, den, b, iw1, ib1, iw2, ib2, tw1, tb1, tw2, tb2, cwp, cwq, cb1)


def _edge_sc_body(as_h, ad_h, ae_h, src_h, dst_h, m_h, hl_h, hr_h, za_h, zd_h,
                  acc_o, den_o,
                  src2, dst2, ae2, av1, dv1, ex1, rowb, mv, acc_s, den_s):
    cid = lax.axis_index("c")
    sid = lax.axis_index("s")

    pltpu.sync_copy(m_h, mv)
    zsl = pl.ds(sid * NSLICE, NSLICE)
    pltpu.sync_copy(za_h, acc_s.at[zsl])

    @pl.when(sid == 0)
    def _():
        pltpu.sync_copy(zd_h, den_s)

    plsc.subcore_barrier()

    mvv = mv[...]
    base_row = sid * TROWS

    def superchunk(ci, carry):
        e0 = (base_row + ci * SUPER) * LANES
        pltpu.sync_copy(src_h.at[pl.ds(e0, SUPER * LANES)], src2)
        pltpu.sync_copy(dst_h.at[pl.ds(e0, SUPER * LANES)], dst2)
        pltpu.sync_copy(ae_h.at[pl.ds(e0, SUPER * LANES)], ae2)
        pltpu.sync_copy(as_h.at[src2], av1)
        pltpu.sync_copy(ad_h.at[dst2], dv1)

        def edge_row(k, c2):
            ksl = pl.ds(k * LANES, LANES)
            @pl.when(cid == 0)
            def _():
                pltpu.sync_copy(hl_h.at[src2.at[ksl]], rowb)

            @pl.when(cid == 1)
            def _():
                pltpu.sync_copy(hr_h.at[src2.at[ksl]], rowb)

            def grp(g, c3):
                sl = pl.ds(k * LANES + g * 16, 16)
                s = av1[sl] + dv1[sl] + ae2[sl]
                a = jnp.maximum(s, 0.2 * s)
                ex1[pl.ds(g * 16, 16)] = jnp.exp(a - mvv)
                return c3

            lax.fori_loop(0, LANES // 16, grp, 0)

            def scale(g, c3):
                lo = pl.ds(0, 16)
                hi = pl.ds(16, 16)
                for r in range(16):
                    j = g * 16 + r
                    sc = plsc.load_gather(ex1, [jnp.full((16,), j, jnp.int32)])
                    rowb[j, lo] = rowb[j, lo] * sc
                    rowb[j, hi] = rowb[j, hi] * sc
                return c3

            lax.fori_loop(0, LANES // 16, scale, 0)

            pltpu.sync_copy(rowb, acc_s.at[dst2.at[ksl]], add=True)

            @pl.when(cid == 0)
            def _():
                pltpu.sync_copy(ex1, den_s.at[dst2.at[ksl]], add=True)
            return c2

        lax.fori_loop(0, SUPER, edge_row, 0)
        return carry

    lax.fori_loop(0, NSUPER, superchunk, 0)
    plsc.subcore_barrier()

    @pl.when(cid == 0)
    def _():
        pltpu.sync_copy(acc_s.at[zsl], acc_o.at[0].at[zsl])

    @pl.when(cid == 1)
    def _():
        pltpu.sync_copy(acc_s.at[zsl], acc_o.at[1].at[zsl])

    @pl.when(jnp.logical_and(cid == 0, sid == 0))
    def _():
        pltpu.sync_copy(den_s, den_o)


_edge_sc = functools.partial(
    pl.kernel,
    _edge_sc_body,
    mesh=_mesh,
    compiler_params=pltpu.CompilerParams(
        needs_layout_passes=False, use_tc_tiling_on_sc=False),
    out_type=[
        jax.ShapeDtypeStruct((2, NPAD, HH), jnp.float32),
        jax.ShapeDtypeStruct((NPAD,), jnp.float32),
    ],
    scratch_types=[
        pltpu.VMEM((SUPER * LANES,), jnp.int32),
        pltpu.VMEM((SUPER * LANES,), jnp.int32),
        pltpu.VMEM((SUPER * LANES,), jnp.float32),
        pltpu.VMEM((SUPER * LANES,), jnp.float32),
        pltpu.VMEM((SUPER * LANES,), jnp.float32),
        pltpu.VMEM((LANES,), jnp.float32),
        pltpu.VMEM((LANES, HH), jnp.float32),
        pltpu.VMEM((16,), jnp.float32),
        pltpu.VMEM_SHARED((NPAD, HH), jnp.float32),
        pltpu.VMEM_SHARED((NPAD,), jnp.float32),
    ],
)


def _xt_sc_body(p_h, q_h, src_h, dst_h, w2_h, b2_h,
                ct_o, wexp, b2v, srcb, dstb, bufp, bufq, outb):
    cid = lax.axis_index("c")
    sid = lax.axis_index("s")
    wid = sid * NC + cid
    base = wid * XTT

    pltpu.sync_copy(w2_h, wexp)
    pltpu.sync_copy(b2_h, b2v)
    b2vec = b2v[...]

    def row(r, carry):
        rr = base + r
        pltpu.sync_copy(src_h.at[pl.ds(rr, 1), :], srcb)
        pltpu.sync_copy(dst_h.at[pl.ds(rr, 1), :], dstb)
        pltpu.sync_copy(p_h.at[srcb.at[0]], bufp)
        pltpu.sync_copy(q_h.at[dstb.at[0]], bufq)
        for g in range(LANES // 16):
            rows16 = lax.iota(jnp.int32, 16) + g * 16

            def fbody(f, acc):
                fi = jnp.full((16,), f, jnp.int32)
                pf = plsc.load_gather(bufp, [rows16, fi])
                qf = plsc.load_gather(bufq, [rows16, fi])
                t = jnp.maximum(pf + qf, 0.0)
                return acc + t * wexp[f, pl.ds(0, 16)]

            acc = lax.fori_loop(0, H, fbody, jnp.zeros((16,), jnp.float32))
            outb[pl.ds(g * 16, 16)] = acc + b2vec
        pltpu.sync_copy(outb, ct_o.at[rr])
        return carry

    lax.fori_loop(0, XTT, row, 0)


_xt_sc = functools.partial(
    pl.kernel,
    _xt_sc_body,
    mesh=_mesh,
    compiler_params=pltpu.CompilerParams(
        needs_layout_passes=False, use_tc_tiling_on_sc=False),
    out_type=jax.ShapeDtypeStruct((XT_ROWS, LANES), jnp.float32),
    scratch_types=[
        pltpu.VMEM((H, 16), jnp.float32),
        pltpu.VMEM((16,), jnp.float32),
        pltpu.VMEM((1, LANES), jnp.int32),
        pltpu.VMEM((1, LANES), jnp.int32),
        pltpu.VMEM((LANES, H), jnp.float32),
        pltpu.VMEM((LANES, H), jnp.float32),
        pltpu.VMEM((LANES,), jnp.float32),
    ],
)


def kernel(x, edge_index, edge_attr, W0, as0, ad0, We0, ae0, b0,
           W1, as1, ad1, We1, ae1, b1, W2, as2, ad2, We2, ae2, b2,
           imp_W1, imp_b1, imp_W2, imp_b2, ct_W1, ct_b1, ct_W2, ct_b2,
           tm_W1, tm_b1, tm_W2, tm_b2):
    src = edge_index[0].astype(jnp.int32)
    dst = edge_index[1].astype(jnp.int32)
    loop = jnp.arange(N, dtype=jnp.int32)
    padi = jnp.zeros((EF - E - N,), jnp.int32)
    srcf = jnp.concatenate([src, loop, padi])
    dstf = jnp.concatenate([dst, loop, padi])

    vw = jnp.stack([We0 @ ae0, We1 @ ae1, We2 @ ae2, jnp.zeros(2)], axis=1)
    ae3, s3, m3 = _edge_alpha(edge_attr, vw.astype(jnp.float32))
    cmean = s3[0] / float(E)

    za = jnp.zeros((NSLICE, HH), jnp.float32)
    zd = jnp.zeros((NPAD,), jnp.float32)
    neg = jnp.full((EF - E - N,), -1e30, jnp.float32)

    def run_layer(proj_out, lidx):
        hl, hr, a_s, a_d, ms, md = proj_out
        mshift = jnp.maximum(0.0, ms[0, 0] + md[0, 0] + m3[0, lidx])
        aef = jnp.concatenate(
            [ae3[:, lidx], jnp.full((N,), cmean[lidx], jnp.float32), neg]
        )
        m16 = jnp.broadcast_to(mshift, (16,)).astype(jnp.float32)
        acc, den = _edge_sc()(a_s.reshape(N), a_d.reshape(N), aef, srcf, dstf,
                              m16, hl, hr, za, zd)
        return acc[:, :N, :], den[:N].reshape(N, 1)

    acc, den = run_layer(_proj0(x, W0, as0.reshape(H, 1), ad0.reshape(H, 1)), 0)
    acc, den = run_layer(
        _proj_mid(acc[0], acc[1], den, b0.reshape(1, H), W1,
                  as1.reshape(H, 1), ad1.reshape(H, 1)), 1)
    acc, den = run_layer(
        _proj_mid(acc[0], acc[1], den, b1.reshape(1, H), W2,
                  as2.reshape(H, 1), ad2.reshape(H, 1)), 2)

    h2 = H // 2
    imp, tm, pp, qq = _final(
        acc[0], acc[1], den, b2.reshape(1, H),
        imp_W1, imp_b1.reshape(1, h2), imp_W2, imp_b2.reshape(1, 1),
        tm_W1, tm_b1.reshape(1, h2), tm_W2, tm_b2.reshape(1, 1),
        ct_W1[:H, :], ct_W1[H:, :], ct_b1.reshape(1, H))

    padx = jnp.zeros((XT_ROWS * LANES - E,), jnp.int32)
    srcx = jnp.concatenate([src, padx]).reshape(XT_ROWS, LANES)
    dstx = jnp.concatenate([dst, padx]).reshape(XT_ROWS, LANES)
    b2c = jnp.broadcast_to(ct_b2.reshape(()), (16,)).astype(jnp.float32)
    w2x = jnp.broadcast_to(ct_W2.reshape(H, 1), (H, 16)).astype(jnp.float32)
    ct = _xt_sc()(pp, qq, srcx, dstx, w2x, b2c)
    crosstalk = ct.reshape(-1)[:E]

    return (imp, crosstalk, tm)


# speedup vs baseline: 12.1634x; 1.0292x over previous
import functools

import jax
import jax.numpy as jnp
from jax import lax
from jax.experimental import pallas as pl
from jax.experimental.pallas import tpu as pltpu
from jax.experimental.pallas import tpu_sc as plsc

N = 50000
E = 800000
H = 64
HH = 32
BLK = 1000
NC = 2
NS = 16
LANES = 128
EF_ROWS = 6656
EF = EF_ROWS * LANES
TROWS = EF_ROWS // NS
SUPER = 16
NSUPER = TROWS // SUPER
XT_ROWS = 6272
XTT = XT_ROWS // (NC * NS)
NPAD = 50048
NSLICE = NPAD // NS

_mesh = plsc.VectorSubcoreMesh(core_axis_name="c", subcore_axis_name="s")


def _relu(v):
    return jnp.maximum(v, 0.0)


def _edge_alpha_body(ea_ref, v_ref, o_ref, s_ref, m_ref):
    i = pl.program_id(0)
    a3 = ea_ref[...] @ v_ref[...]

    @pl.when(i == 0)
    def _():
        s_ref[...] = jnp.zeros_like(s_ref)
        m_ref[...] = jnp.full_like(m_ref, -1e30)

    o_ref[...] = a3
    s_ref[...] = s_ref[...] + jnp.sum(a3, axis=0, keepdims=True)
    m_ref[...] = jnp.maximum(m_ref[...], jnp.max(a3, axis=0, keepdims=True))


def _edge_alpha(ea, vw):
    grid = (E // BLK,)
    return pl.pallas_call(
        _edge_alpha_body,
        grid=grid,
        in_specs=[
            pl.BlockSpec((BLK, 2), lambda i: (i, 0)),
            pl.BlockSpec((2, 4), lambda i: (0, 0)),
        ],
        out_specs=[
            pl.BlockSpec((BLK, 4), lambda i: (i, 0)),
            pl.BlockSpec((1, 4), lambda i: (0, 0)),
            pl.BlockSpec((1, 4), lambda i: (0, 0)),
        ],
        out_shape=[
            jax.ShapeDtypeStruct((E, 4), jnp.float32),
            jax.ShapeDtypeStruct((1, 4), jnp.float32),
            jax.ShapeDtypeStruct((1, 4), jnp.float32),
        ],
    )(ea, vw)


def _proj_tail(h, as_ref, ad_ref, hl_ref, hr_ref, a_ref, d_ref, ms_ref, md_ref, i):
    hl_ref[...] = h[:, :HH]
    hr_ref[...] = h[:, HH:]
    als = h @ as_ref[...]
    ald = h @ ad_ref[...]
    a_ref[...] = als
    d_ref[...] = ald

    @pl.when(i == 0)
    def _():
        ms_ref[...] = jnp.full_like(ms_ref, -1e30)
        md_ref[...] = jnp.full_like(md_ref, -1e30)

    ms_ref[...] = jnp.maximum(ms_ref[...], jnp.max(als, axis=0, keepdims=True))
    md_ref[...] = jnp.maximum(md_ref[...], jnp.max(ald, axis=0, keepdims=True))


def _proj0_body(x_ref, w_ref, as_ref, ad_ref,
                hl_ref, hr_ref, a_ref, d_ref, ms_ref, md_ref):
    h = x_ref[...] @ w_ref[...]
    _proj_tail(h, as_ref, ad_ref, hl_ref, hr_ref, a_ref, d_ref, ms_ref, md_ref,
               pl.program_id(0))


def _proj_mid_body(al_ref, ar_ref, dn_ref, b_ref, w_ref, as_ref, ad_ref,
                   hl_ref, hr_ref, a_ref, d_ref, ms_ref, md_ref):
    invd = 1.0 / (dn_ref[...] + 1e-16)
    xl = _relu(al_ref[...] * invd + b_ref[:, :HH])
    xr = _relu(ar_ref[...] * invd + b_ref[:, HH:])
    h = xl @ w_ref[:HH, :] + xr @ w_ref[HH:, :]
    _proj_tail(h, as_ref, ad_ref, hl_ref, hr_ref, a_ref, d_ref, ms_ref, md_ref,
               pl.program_id(0))


def _proj_outs():
    return (
        [
            pl.BlockSpec((BLK, HH), lambda i: (i, 0)),
            pl.BlockSpec((BLK, HH), lambda i: (i, 0)),
            pl.BlockSpec((BLK, 1), lambda i: (i, 0)),
            pl.BlockSpec((BLK, 1), lambda i: (i, 0)),
            pl.BlockSpec((1, 1), lambda i: (0, 0)),
            pl.BlockSpec((1, 1), lambda i: (0, 0)),
        ],
        [
            jax.ShapeDtypeStruct((N, HH), jnp.float32),
            jax.ShapeDtypeStruct((N, HH), jnp.float32),
            jax.ShapeDtypeStruct((N, 1), jnp.float32),
            jax.ShapeDtypeStruct((N, 1), jnp.float32),
            jax.ShapeDtypeStruct((1, 1), jnp.float32),
            jax.ShapeDtypeStruct((1, 1), jnp.float32),
        ],
    )


def _proj0(x, w, a_s, a_d):
    grid = (N // BLK,)
    outs, shapes = _proj_outs()
    return pl.pallas_call(
        _proj0_body,
        grid=grid,
        in_specs=[
            pl.BlockSpec((BLK, 4), lambda i: (i, 0)),
            pl.BlockSpec((4, H), lambda i: (0, 0)),
            pl.BlockSpec((H, 1), lambda i: (0, 0)),
            pl.BlockSpec((H, 1), lambda i: (0, 0)),
        ],
        out_specs=outs,
        out_shape=shapes,
    )(x, w, a_s, a_d)


def _proj_mid(accl, accr, den, b, w, a_s, a_d):
    grid = (N // BLK,)
    outs, shapes = _proj_outs()
    return pl.pallas_call(
        _proj_mid_body,
        grid=grid,
        in_specs=[
            pl.BlockSpec((BLK, HH), lambda i: (i, 0)),
            pl.BlockSpec((BLK, HH), lambda i: (i, 0)),
            pl.BlockSpec((BLK, 1), lambda i: (i, 0)),
            pl.BlockSpec((1, H), lambda i: (0, 0)),
            pl.BlockSpec((H, H), lambda i: (0, 0)),
            pl.BlockSpec((H, 1), lambda i: (0, 0)),
            pl.BlockSpec((H, 1), lambda i: (0, 0)),
        ],
        out_specs=outs,
        out_shape=shapes,
    )(accl, accr, den, b, w, a_s, a_d)


def _final_body(al_ref, ar_ref, dn_ref, b_ref,
                iw1_ref, ib1_ref, iw2_ref, ib2_ref,
                tw1_ref, tb1_ref, tw2_ref, tb2_ref,
                cwp_ref, cwq_ref, cb1_ref,
                imp_ref, tm_ref, p_ref, q_ref):
    invd = 1.0 / (dn_ref[...] + 1e-16)
    hl = al_ref[...] * invd + b_ref[:, :HH]
    hr = ar_ref[...] * invd + b_ref[:, HH:]
    h = jnp.concatenate([hl, hr], axis=1)
    imp_ref[...] = _relu(h @ iw1_ref[...] + ib1_ref[...]) @ iw2_ref[...] + ib2_ref[...]
    tm_ref[...] = _relu(h @ tw1_ref[...] + tb1_ref[...]) @ tw2_ref[...] + tb2_ref[...]
    p_ref[...] = h @ cwp_ref[...] + cb1_ref[...]
    q_ref[...] = h @ cwq_ref[...]


def _final(accl, accr, den, b, iw1, ib1, iw2, ib2, tw1, tb1, tw2, tb2,
           cwp, cwq, cb1):
    grid = (N // BLK,)
    h2 = H // 2
    return pl.pallas_call(
        _final_body,
        grid=grid,
        in_specs=[
            pl.BlockSpec((BLK, HH), lambda i: (i, 0)),
            pl.BlockSpec((BLK, HH), lambda i: (i, 0)),
            pl.BlockSpec((BLK, 1), lambda i: (i, 0)),
            pl.BlockSpec((1, H), lambda i: (0, 0)),
            pl.BlockSpec((H, h2), lambda i: (0, 0)),
            pl.BlockSpec((1, h2), lambda i: (0, 0)),
            pl.BlockSpec((h2, 1), lambda i: (0, 0)),
            pl.BlockSpec((1, 1), lambda i: (0, 0)),
            pl.BlockSpec((H, h2), lambda i: (0, 0)),
            pl.BlockSpec((1, h2), lambda i: (0, 0)),
            pl.BlockSpec((h2, 1), lambda i: (0, 0)),
            pl.BlockSpec((1, 1), lambda i: (0, 0)),
            pl.BlockSpec((H, H), lambda i: (0, 0)),
            pl.BlockSpec((H, H), lambda i: (0, 0)),
            pl.BlockSpec((1, H), lambda i: (0, 0)),
        ],
        out_specs=[
            pl.BlockSpec((BLK, 1), lambda i: (i, 0)),
            pl.BlockSpec((BLK, 1), lambda i: (i, 0)),
            pl.BlockSpec((BLK, H), lambda i: (i, 0)),
            pl.BlockSpec((BLK, H), lambda i: (i, 0)),
        ],
        out_shape=[
            jax.ShapeDtypeStruct((N, 1), jnp.float32),
            jax.ShapeDtypeStruct((N, 1), jnp.float32),
            jax.ShapeDtypeStruct((N, H), jnp.float32),
            jax.ShapeDtypeStruct((N, H), jnp.float32),
        ],
    )(accl, accr, den, b, iw1, ib1, iw2, ib2, tw1, tb1, tw2, tb2, cwp, cwq, cb1)


def _edge_sc_body(as_h, ad_h, ae_h, src_h, dst_h, m_h, hl_h, hr_h, za_h, zd_h,
                  acc_o, den_o,
                  src2, dst2, ae2, av1, dv1, ex1, rowb, mv, acc_s, den_s):
    cid = lax.axis_index("c")
    sid = lax.axis_index("s")

    pltpu.sync_copy(m_h, mv)
    zsl = pl.ds(sid * NSLICE, NSLICE)
    pltpu.sync_copy(za_h, acc_s.at[zsl])

    @pl.when(sid == 0)
    def _():
        pltpu.sync_copy(zd_h, den_s)

    plsc.subcore_barrier()

    mvv = mv[...]
    base_row = sid * TROWS

    def superchunk(ci, carry):
        e0 = (base_row + ci * SUPER) * LANES
        pltpu.sync_copy(src_h.at[pl.ds(e0, SUPER * LANES)], src2)
        pltpu.sync_copy(dst_h.at[pl.ds(e0, SUPER * LANES)], dst2)
        pltpu.sync_copy(ae_h.at[pl.ds(e0, SUPER * LANES)], ae2)
        pltpu.sync_copy(as_h.at[src2], av1)
        pltpu.sync_copy(ad_h.at[dst2], dv1)

        def edge_row(k, c2):
            ksl = pl.ds(k * LANES, LANES)
            @pl.when(cid == 0)
            def _():
                pltpu.sync_copy(hl_h.at[src2.at[ksl]], rowb)

            @pl.when(cid == 1)
            def _():
                pltpu.sync_copy(hr_h.at[src2.at[ksl]], rowb)

            def grp(g, c3):
                sl = pl.ds(k * LANES + g * 16, 16)
                s = av1[sl] + dv1[sl] + ae2[sl]
                a = jnp.maximum(s, 0.2 * s)
                ex1[pl.ds(g * 16, 16)] = jnp.exp(a - mvv)
                return c3

            lax.fori_loop(0, LANES // 16, grp, 0)

            def scale(g, c3):
                lo = pl.ds(0, 16)
                hi = pl.ds(16, 16)
                for r in range(16):
                    j = g * 16 + r
                    sc = plsc.load_gather(ex1, [jnp.full((16,), j, jnp.int32)])
                    rowb[j, lo] = rowb[j, lo] * sc
                    rowb[j, hi] = rowb[j, hi] * sc
                return c3

            lax.fori_loop(0, LANES // 16, scale, 0)

            pltpu.sync_copy(rowb, acc_s.at[dst2.at[ksl]], add=True)

            @pl.when(cid == 0)
            def _():
                pltpu.sync_copy(ex1, den_s.at[dst2.at[ksl]], add=True)
            return c2

        lax.fori_loop(0, SUPER, edge_row, 0)
        return carry

    lax.fori_loop(0, NSUPER, superchunk, 0)
    plsc.subcore_barrier()

    @pl.when(cid == 0)
    def _():
        pltpu.sync_copy(acc_s.at[zsl], acc_o.at[0].at[zsl])

    @pl.when(cid == 1)
    def _():
        pltpu.sync_copy(acc_s.at[zsl], acc_o.at[1].at[zsl])

    @pl.when(jnp.logical_and(cid == 0, sid == 0))
    def _():
        pltpu.sync_copy(den_s, den_o)


_edge_sc = functools.partial(
    pl.kernel,
    _edge_sc_body,
    mesh=_mesh,
    compiler_params=pltpu.CompilerParams(
        needs_layout_passes=False, use_tc_tiling_on_sc=False),
    out_type=[
        jax.ShapeDtypeStruct((2, NPAD, HH), jnp.float32),
        jax.ShapeDtypeStruct((NPAD,), jnp.float32),
    ],
    scratch_types=[
        pltpu.VMEM((SUPER * LANES,), jnp.int32),
        pltpu.VMEM((SUPER * LANES,), jnp.int32),
        pltpu.VMEM((SUPER * LANES,), jnp.float32),
        pltpu.VMEM((SUPER * LANES,), jnp.float32),
        pltpu.VMEM((SUPER * LANES,), jnp.float32),
        pltpu.VMEM((LANES,), jnp.float32),
        pltpu.VMEM((LANES, HH), jnp.float32),
        pltpu.VMEM((16,), jnp.float32),
        pltpu.VMEM_SHARED((NPAD, HH), jnp.float32),
        pltpu.VMEM_SHARED((NPAD,), jnp.float32),
    ],
)


def _xt_sc_body(p_h, q_h, src_h, dst_h, w2_h, b2_h,
                ct_o, wexp, b2v, srcb, dstb, bufp, bufq, outb):
    cid = lax.axis_index("c")
    sid = lax.axis_index("s")
    wid = sid * NC + cid
    base = wid * XTT

    pltpu.sync_copy(w2_h, wexp)
    pltpu.sync_copy(b2_h, b2v)
    b2vec = b2v[...]

    def row(r, carry):
        rr = base + r
        pltpu.sync_copy(src_h.at[pl.ds(rr, 1), :], srcb)
        pltpu.sync_copy(dst_h.at[pl.ds(rr, 1), :], dstb)
        pltpu.sync_copy(p_h.at[srcb.at[0]], bufp)
        pltpu.sync_copy(q_h.at[dstb.at[0]], bufq)
        for g in range(LANES // 16):
            rows16 = lax.iota(jnp.int32, 16) + g * 16

            def fbody(f, acc):
                fi = jnp.full((16,), f, jnp.int32)
                pf = plsc.load_gather(bufp, [rows16, fi])
                qf = plsc.load_gather(bufq, [rows16, fi])
                t = jnp.maximum(pf + qf, 0.0)
                return acc + t * wexp[f, pl.ds(0, 16)]

            acc = lax.fori_loop(0, H, fbody, jnp.zeros((16,), jnp.float32))
            outb[pl.ds(g * 16, 16)] = acc + b2vec
        pltpu.sync_copy(outb, ct_o.at[rr])
        return carry

    lax.fori_loop(0, XTT, row, 0)


_xt_sc = functools.partial(
    pl.kernel,
    _xt_sc_body,
    mesh=_mesh,
    compiler_params=pltpu.CompilerParams(
        needs_layout_passes=False, use_tc_tiling_on_sc=False),
    out_type=jax.ShapeDtypeStruct((XT_ROWS, LANES), jnp.float32),
    scratch_types=[
        pltpu.VMEM((H, 16), jnp.float32),
        pltpu.VMEM((16,), jnp.float32),
        pltpu.VMEM((1, LANES), jnp.int32),
        pltpu.VMEM((1, LANES), jnp.int32),
        pltpu.VMEM((LANES, H), jnp.float32),
        pltpu.VMEM((LANES, H), jnp.float32),
        pltpu.VMEM((LANES,), jnp.float32),
    ],
)


def kernel(x, edge_index, edge_attr, W0, as0, ad0, We0, ae0, b0,
           W1, as1, ad1, We1, ae1, b1, W2, as2, ad2, We2, ae2, b2,
           imp_W1, imp_b1, imp_W2, imp_b2, ct_W1, ct_b1, ct_W2, ct_b2,
           tm_W1, tm_b1, tm_W2, tm_b2):
    src = edge_index[0].astype(jnp.int32)
    dst = edge_index[1].astype(jnp.int32)
    loop = jnp.arange(N, dtype=jnp.int32)
    padi = jnp.zeros((EF - E - N,), jnp.int32)
    srcf = jnp.concatenate([src, loop, padi])
    dstf = jnp.concatenate([dst, loop, padi])

    vw = jnp.stack([We0 @ ae0, We1 @ ae1, We2 @ ae2, jnp.zeros(2)], axis=1)
    ae3, s3, m3 = _edge_alpha(edge_attr, vw.astype(jnp.float32))
    cmean = s3[0] / float(E)

    za = jnp.zeros((NSLICE, HH), jnp.float32)
    zd = jnp.zeros((NPAD,), jnp.float32)
    neg = jnp.full((EF - E - N,), -1e30, jnp.float32)

    def run_layer(proj_out, lidx):
        hl, hr, a_s, a_d, ms, md = proj_out
        mshift = jnp.maximum(0.0, ms[0, 0] + md[0, 0] + m3[0, lidx])
        aef = jnp.concatenate(
            [ae3[:, lidx], jnp.full((N,), cmean[lidx], jnp.float32), neg]
        )
        m16 = jnp.broadcast_to(mshift, (16,)).astype(jnp.float32)
        acc, den = _edge_sc()(a_s.reshape(N), a_d.reshape(N), aef, srcf, dstf,
                              m16, hl, hr, za, zd)
        return acc[:, :N, :], den[:N].reshape(N, 1)

    acc, den = run_layer(_proj0(x, W0, as0.reshape(H, 1), ad0.reshape(H, 1)), 0)
    acc, den = run_layer(
        _proj_mid(acc[0], acc[1], den, b0.reshape(1, H), W1,
                  as1.reshape(H, 1), ad1.reshape(H, 1)), 1)
    acc, den = run_layer(
        _proj_mid(acc[0], acc[1], den, b1.reshape(1, H), W2,
                  as2.reshape(H, 1), ad2.reshape(H, 1)), 2)

    h2 = H // 2
    imp, tm, pp, qq = _final(
        acc[0], acc[1], den, b2.reshape(1, H),
        imp_W1, imp_b1.reshape(1, h2), imp_W2, imp_b2.reshape(1, 1),
        tm_W1, tm_b1.reshape(1, h2), tm_W2, tm_b2.reshape(1, 1),
        ct_W1[:H, :], ct_W1[H:, :], ct_b1.reshape(1, H))

    padx = jnp.zeros((XT_ROWS * LANES - E,), jnp.int32)
    srcx = jnp.concatenate([src, padx]).reshape(XT_ROWS, LANES)
    dstx = jnp.concatenate([dst, padx]).reshape(XT_ROWS, LANES)
    b2c = jnp.broadcast_to(ct_b2.reshape(()), (16,)).astype(jnp.float32)
    w2x = jnp.broadcast_to(ct_W2.reshape(H, 1), (H, 16)).astype(jnp.float32)
    ct = _xt_sc()(pp, qq, srcx, dstx, w2x, b2c)
    crosstalk = ct.reshape(-1)[:E]

    return (imp, crosstalk, tm)
